# baseline (speedup 1.0000x reference)
import functools

import jax
import jax.numpy as jnp
from jax import lax
from jax.experimental import pallas as pl
from jax.experimental.pallas import tpu as pltpu
from jax.experimental.pallas import tpu_sc as plsc

_B, _CIN, _COUT, _D = 64, 256, 128, 1024
_NC, _NS, _L = 2, 16, 16
_NW = _NC * _NS
_BPW = _B // _NW
_CH = 16
_CPB = _COUT // _CH
_NCHUNK = _BPW * _CPB
_NBUF = 7

_mesh = plsc.VectorSubcoreMesh(core_axis_name="c", subcore_axis_name="s")


@functools.partial(
    pl.kernel,
    mesh=_mesh,
    out_type=jax.ShapeDtypeStruct((_B, _COUT, _D), jnp.float32),
    scratch_types=[
        pltpu.VMEM((_COUT,), jnp.int32),
        pltpu.VMEM((_NBUF * _CH, _D), jnp.float32),
        pltpu.SemaphoreType.DMA((_NBUF,)),
        pltpu.SemaphoreType.DMA((_NBUF,)),
    ],
)
def _gather(x_hbm, idx_hbm, out_hbm, tab_v, ring_v, gsem, ssem):
    wid = lax.axis_index("s") * _NC + lax.axis_index("c")
    b0 = wid * _BPW
    pltpu.sync_copy(idx_hbm, tab_v)

    def _slot(ci):
        return lax.rem(ci, _NBUF)

    def _bj(ci):
        bi = ci // _CPB
        return b0 + bi, (ci - bi * _CPB) * _CH

    def _gather_copy(ci):
        p = _slot(ci)
        b, j0 = _bj(ci)
        return pltpu.make_async_copy(
            x_hbm.at[b].at[tab_v.at[pl.ds(j0, _CH)]],
            ring_v.at[pl.ds(p * _CH, _CH)],
            gsem.at[p],
        )

    def _scatter_copy(ci):
        p = _slot(ci)
        b, j0 = _bj(ci)
        return pltpu.make_async_copy(
            ring_v.at[pl.ds(p * _CH, _CH)],
            out_hbm.at[b].at[pl.ds(j0, _CH)],
            ssem.at[p],
        )

    def prime(ci, carry):
        _gather_copy(ci).start()
        return carry

    lax.fori_loop(0, _NBUF, prime, 0)

    def body(ci, carry):
        _gather_copy(ci).wait()
        _scatter_copy(ci).start()

        @pl.when(ci + _NBUF < _NCHUNK)
        def _():
            _scatter_copy(ci).wait()
            _gather_copy(ci + _NBUF).start()

        return carry

    lax.fori_loop(0, _NCHUNK, body, 0)

    def drain(ci, carry):
        _scatter_copy(ci).wait()
        return carry

    lax.fori_loop(_NCHUNK - _NBUF, _NCHUNK, drain, 0)


def kernel(x, indices):
    return _gather(x, indices)
